# baseline (speedup 1.0000x reference)
import functools

import jax
import jax.numpy as jnp
from jax import lax
from jax.experimental import pallas as pl
from jax.experimental.pallas import tpu as pltpu
from jax.experimental.pallas import tpu_sc as plsc

_L = 16
_NS = 16


@jax.jit
def _extract_eos_sc(tokens, mask):
    B, N, D = tokens.shape

    mesh = plsc.VectorSubcoreMesh(
        core_axis_name="c", subcore_axis_name="s", num_cores=1, num_subcores=_NS
    )

    @functools.partial(
        pl.kernel,
        out_type=jax.ShapeDtypeStruct((B, D), tokens.dtype),
        mesh=mesh,
        scratch_types=[
            pltpu.VMEM((N,), jnp.int32),
            pltpu.VMEM((1, D), jnp.float32),
        ],
        compiler_params=pltpu.CompilerParams(needs_layout_passes=False),
    )
    def k(tokens_hbm, mask_hbm, out_hbm, mask_v, row_v):
        b = lax.axis_index("s") + lax.axis_index("c")

        @pl.when(b < B)
        def _():
            pltpu.sync_copy(mask_hbm.at[b], mask_v)
            lane = lax.iota(jnp.int32, _L)
            big = jnp.int32(N)

            def cond(carry):
                c, found = carry
                return (found >= big) & (c < N // _L)

            def body(carry):
                c, _ = carry
                chunk = mask_v[pl.ds(c * _L, _L)]
                cand = jnp.min(jnp.where(chunk != 0, c * _L + lane, big))
                return c + 1, cand

            _, found = lax.while_loop(cond, body, (jnp.int32(0), big))
            idx = jnp.where(found >= big, 0, found)
            pltpu.sync_copy(tokens_hbm.at[b, pl.ds(idx, 1), :], row_v)
            pltpu.sync_copy(row_v, out_hbm.at[pl.ds(b, 1), :])

    return k(tokens, mask)


def kernel(tokens, eos_token_mask):
    return _extract_eos_sc(tokens, eos_token_mask)
